# baseline (speedup 1.0000x reference)
import functools

import jax
import jax.numpy as jnp
from jax import lax
from jax.experimental import pallas as pl
from jax.experimental.pallas import tpu as pltpu
from jax.experimental.pallas import tpu_sc as plsc

_Q = 256
_N = 100000
_D = 128
_M = 10
_K = 10
_TOP_X = 3

_BN = 2048
_NB = (_N + _BN - 1) // _BN
_NPAD = _NB * _BN
_RT = 2560
_NEG = -3.0e38
_IMAX = 2**31 - 1
_PREC = lax.Precision.DEFAULT
_PREC_ACC = lax.Precision.HIGHEST


def _topk_extract(s, col, k):
    vs, is_ = [], []
    for _ in range(k):
        m = jnp.max(s, axis=1, keepdims=True)
        im = jnp.min(jnp.where(s == m, col, _IMAX), axis=1, keepdims=True)
        s = jnp.where(col == im, _NEG, s)
        vs.append(m)
        is_.append(im)
    return jnp.concatenate(vs, axis=1), jnp.concatenate(is_, axis=1)


def _stage1_body(q_ref, db_ref, ids_ref, runv_ref, runi_ref):
    b = pl.program_id(0)
    s = lax.dot_general(q_ref[...], db_ref[...], (((1,), (1,)), ((), ())),
                        precision=_PREC, preferred_element_type=jnp.float32)
    col = lax.broadcasted_iota(jnp.int32, (_Q, _BN), 1) + b * _BN
    s = jnp.where(col < _N, s, _NEG)

    @pl.when(b == 0)
    def _():
        runv_ref[...] = jnp.full((_Q, _M), _NEG, jnp.float32)
        runi_ref[...] = jnp.full((_Q, _M), _IMAX, jnp.int32)

    h = _BN // 2
    a, b2 = s[:, :h], s[:, h:]
    ca, cb = col[:, :h], col[:, h:]
    sel_a = a >= b2
    p = jnp.where(sel_a, a, b2)
    pm = jnp.where(sel_a, b2, a)
    pi = jnp.where(sel_a, ca, cb)
    pmi = jnp.where(sel_a, cb, ca)
    bv, bi = [], []
    for _ in range(_M):
        m = jnp.max(p, axis=1, keepdims=True)
        im = jnp.min(jnp.where(p == m, pi, _IMAX), axis=1, keepdims=True)
        sel = pi == im
        p = jnp.where(sel, pm, p)
        pi = jnp.where(sel, pmi, pi)
        pm = jnp.where(sel, _NEG, pm)
        bv.append(m)
        bi.append(im)
    cv = jnp.concatenate([runv_ref[...]] + bv, axis=1)
    ci = jnp.concatenate([runi_ref[...]] + bi, axis=1)
    nv, ni = _topk_extract(cv, ci, _M)
    runv_ref[...] = nv
    runi_ref[...] = ni
    ids_ref[...] = ni


def _stage2_tau_body(g_ref, db_ref, tau_ref, runv_ref):
    b = pl.program_id(1)
    s = lax.dot_general(g_ref[...], db_ref[...], (((1,), (1,)), ((), ())),
                        precision=_PREC, preferred_element_type=jnp.float32)
    col = lax.broadcasted_iota(jnp.int32, (_RT, _BN), 1) + b * _BN
    s = jnp.where(col < _N, s, _NEG)

    @pl.when(b == 0)
    def _():
        runv_ref[...] = jnp.full((_RT, _K - 1), _NEG, jnp.float32)

    h = _BN // 2
    a, b2 = s[:, :h], s[:, h:]
    p = jnp.maximum(a, b2)
    pm = jnp.minimum(a, b2)
    bv = []
    for _ in range(_K - 1):
        m = jnp.max(p, axis=1, keepdims=True)
        sel = p == m
        p = jnp.where(sel, pm, p)
        pm = jnp.where(sel, _NEG, pm)
        bv.append(m)
    cv = jnp.concatenate([runv_ref[...]] + bv, axis=1)
    nv = []
    for _ in range(_K - 1):
        m = jnp.max(cv, axis=1, keepdims=True)
        cv = jnp.where(cv == m, _NEG, cv)
        nv.append(m)
    runv_ref[...] = jnp.concatenate(nv, axis=1)
    tau_ref[...] = runv_ref[:, _K - 2:_K - 1]


def _stage2_agg_body(g_ref, db_ref, tau_ref, w_ref):
    b = pl.program_id(1)
    s = lax.dot_general(g_ref[...], db_ref[...], (((1,), (1,)), ((), ())),
                        precision=_PREC, preferred_element_type=jnp.float32)
    col = lax.broadcasted_iota(jnp.int32, (_RT, _BN), 1) + b * _BN
    sm = jnp.where((s >= tau_ref[...]) & (col < _N), s, 0.0)
    p = lax.dot_general(sm, db_ref[...], (((1,), (0,)), ((), ())),
                        precision=_PREC_ACC,
                        preferred_element_type=jnp.float32)

    @pl.when(b == 0)
    def _():
        w_ref[...] = jnp.zeros((_RT, _D), jnp.float32)

    w_ref[...] += p


_NW = 32
_RPW = _Q * _M // _NW


@functools.partial(
    pl.kernel,
    mesh=plsc.VectorSubcoreMesh(core_axis_name="c", subcore_axis_name="s"),
    out_type=jax.ShapeDtypeStruct((_Q * _M, _D), jnp.float32),
    scratch_types=[pltpu.VMEM((_RPW,), jnp.int32),
                   pltpu.VMEM((_RPW, _D), jnp.float32),
                   pltpu.SemaphoreType.DMA],
)
def _sc_gather(idx_hbm, db_hbm, out_hbm, idx_v, rows_v, sem):
    wid = lax.axis_index("s") * 2 + lax.axis_index("c")
    base = wid * _RPW
    pltpu.sync_copy(idx_hbm.at[pl.ds(base, _RPW)], idx_v)
    pltpu.async_copy(db_hbm.at[idx_v], rows_v, sem).wait()
    pltpu.sync_copy(rows_v, out_hbm.at[pl.ds(base, _RPW)])


def _final_body(q_ref, g_ref, w2_ref, ids_ref, selids_ref, scores_ref):
    q = q_ref[...]
    t = q
    for j in range(_K - 1):
        t = jnp.maximum(t, g_ref[:, j, :])
    tn = t * lax.rsqrt(jnp.maximum(jnp.sum(t * t, axis=1, keepdims=True),
                                   1e-24))
    def _b(x):
        return x.astype(jnp.bfloat16).astype(jnp.float32)
    qb = _b(q)
    tb = _b(tn)
    svals = []
    for mi in range(_M):
        w = q + w2_ref[:, mi, :]
        rn = w * lax.rsqrt(jnp.maximum(jnp.sum(w * w, axis=1, keepdims=True),
                                       1e-24))
        rb = _b(rn)
        s1 = jnp.sum(rb * qb, axis=1, keepdims=True)
        s2 = jnp.sum(rb * tb, axis=1, keepdims=True)
        svals.append(0.5 * (s1 + s2))
    sc = jnp.concatenate(svals, axis=1)
    colm = lax.broadcasted_iota(jnp.int32, (_Q, _M), 1)
    ids = ids_ref[...]
    outv, outi = [], []
    for _ in range(_TOP_X):
        m = jnp.max(sc, axis=1, keepdims=True)
        im = jnp.min(jnp.where(sc == m, colm, _IMAX), axis=1, keepdims=True)
        sel = jnp.sum(jnp.where(colm == im, ids, 0), axis=1, keepdims=True)
        sc = jnp.where(colm == im, _NEG, sc)
        outv.append(m)
        outi.append(sel)
    scores_ref[...] = jnp.concatenate(outv, axis=1)
    selids_ref[...] = jnp.concatenate(outi, axis=1)


def kernel(query_features, db_keys, db_ids):
    del db_ids
    q = query_features
    db = jnp.pad(db_keys, ((0, _NPAD - _N), (0, 0)))

    ids = pl.pallas_call(
        _stage1_body,
        grid=(_NB,),
        in_specs=[
            pl.BlockSpec((_Q, _D), lambda b: (0, 0)),
            pl.BlockSpec((_BN, _D), lambda b: (b, 0)),
        ],
        out_specs=pl.BlockSpec((_Q, _M), lambda b: (0, 0)),
        out_shape=jax.ShapeDtypeStruct((_Q, _M), jnp.int32),
        scratch_shapes=[pltpu.VMEM((_Q, _M), jnp.float32),
                        pltpu.VMEM((_Q, _M), jnp.int32)],
    )(q, db)

    flat_ids = ids.reshape(_Q * _M)
    g = _sc_gather(flat_ids, db_keys)

    nrt = _Q * _M // _RT
    tau = pl.pallas_call(
        _stage2_tau_body,
        grid=(nrt, _NB),
        in_specs=[
            pl.BlockSpec((_RT, _D), lambda r, b: (r, 0)),
            pl.BlockSpec((_BN, _D), lambda r, b: (b, 0)),
        ],
        out_specs=pl.BlockSpec((_RT, 1), lambda r, b: (r, 0)),
        out_shape=jax.ShapeDtypeStruct((_Q * _M, 1), jnp.float32),
        scratch_shapes=[pltpu.VMEM((_RT, _K - 1), jnp.float32)],
    )(g, db)

    w2 = pl.pallas_call(
        _stage2_agg_body,
        grid=(nrt, _NB),
        in_specs=[
            pl.BlockSpec((_RT, _D), lambda r, b: (r, 0)),
            pl.BlockSpec((_BN, _D), lambda r, b: (b, 0)),
            pl.BlockSpec((_RT, 1), lambda r, b: (r, 0)),
        ],
        out_specs=pl.BlockSpec((_RT, _D), lambda r, b: (r, 0)),
        out_shape=jax.ShapeDtypeStruct((_Q * _M, _D), jnp.float32),
    )(g, db, tau)

    selids, scores = pl.pallas_call(
        _final_body,
        in_specs=[
            pl.BlockSpec((_Q, _D), lambda: (0, 0)),
            pl.BlockSpec((_Q, _M, _D), lambda: (0, 0, 0)),
            pl.BlockSpec((_Q, _M, _D), lambda: (0, 0, 0)),
            pl.BlockSpec((_Q, _M), lambda: (0, 0)),
        ],
        out_specs=[pl.BlockSpec((_Q, _TOP_X), lambda: (0, 0)),
                   pl.BlockSpec((_Q, _TOP_X), lambda: (0, 0))],
        out_shape=[jax.ShapeDtypeStruct((_Q, _TOP_X), jnp.int32),
                   jax.ShapeDtypeStruct((_Q, _TOP_X), jnp.float32)],
    )(q, g.reshape(_Q, _M, _D), w2.reshape(_Q, _M, _D), ids)

    return selids, scores


# speedup vs baseline: 1.0023x; 1.0023x over previous
import functools

import jax
import jax.numpy as jnp
from jax import lax
from jax.experimental import pallas as pl
from jax.experimental.pallas import tpu as pltpu
from jax.experimental.pallas import tpu_sc as plsc

_Q = 256
_N = 100000
_D = 128
_M = 10
_K = 10
_TOP_X = 3

_BN = 2048
_NB = (_N + _BN - 1) // _BN
_NPAD = _NB * _BN
_RT = 1280
_NEG = -3.0e38
_IMAX = 2**31 - 1
_PREC = lax.Precision.DEFAULT
_PREC_ACC = lax.Precision.HIGHEST


def _topk_extract(s, col, k):
    vs, is_ = [], []
    for _ in range(k):
        m = jnp.max(s, axis=1, keepdims=True)
        im = jnp.min(jnp.where(s == m, col, _IMAX), axis=1, keepdims=True)
        s = jnp.where(col == im, _NEG, s)
        vs.append(m)
        is_.append(im)
    return jnp.concatenate(vs, axis=1), jnp.concatenate(is_, axis=1)


def _stage1_body(q_ref, db_ref, ids_ref, runv_ref, runi_ref):
    b = pl.program_id(0)
    s = lax.dot_general(q_ref[...], db_ref[...], (((1,), (1,)), ((), ())),
                        precision=_PREC, preferred_element_type=jnp.float32)
    col = lax.broadcasted_iota(jnp.int32, (_Q, _BN), 1) + b * _BN
    s = jnp.where(col < _N, s, _NEG)

    @pl.when(b == 0)
    def _():
        runv_ref[...] = jnp.full((_Q, _M), _NEG, jnp.float32)
        runi_ref[...] = jnp.full((_Q, _M), _IMAX, jnp.int32)

    h = _BN // 2
    a, b2 = s[:, :h], s[:, h:]
    ca, cb = col[:, :h], col[:, h:]
    sel_a = a >= b2
    p = jnp.where(sel_a, a, b2)
    pm = jnp.where(sel_a, b2, a)
    pi = jnp.where(sel_a, ca, cb)
    pmi = jnp.where(sel_a, cb, ca)
    bv, bi = [], []
    for _ in range(_M):
        m = jnp.max(p, axis=1, keepdims=True)
        im = jnp.min(jnp.where(p == m, pi, _IMAX), axis=1, keepdims=True)
        sel = pi == im
        p = jnp.where(sel, pm, p)
        pi = jnp.where(sel, pmi, pi)
        pm = jnp.where(sel, _NEG, pm)
        bv.append(m)
        bi.append(im)
    cv = jnp.concatenate([runv_ref[...]] + bv, axis=1)
    ci = jnp.concatenate([runi_ref[...]] + bi, axis=1)
    nv, ni = _topk_extract(cv, ci, _M)
    runv_ref[...] = nv
    runi_ref[...] = ni
    ids_ref[...] = ni


def _stage2_tau_body(g_ref, db_ref, tau_ref, runv_ref):
    b = pl.program_id(1)
    s = lax.dot_general(g_ref[...], db_ref[...], (((1,), (1,)), ((), ())),
                        precision=_PREC, preferred_element_type=jnp.float32)
    col = lax.broadcasted_iota(jnp.int32, (_RT, _BN), 1) + b * _BN
    s = jnp.where(col < _N, s, _NEG)

    @pl.when(b == 0)
    def _():
        runv_ref[...] = jnp.full((_RT, _K - 1), _NEG, jnp.float32)

    h = _BN // 2
    a, b2 = s[:, :h], s[:, h:]
    p = jnp.maximum(a, b2)
    pm = jnp.minimum(a, b2)
    bv = []
    for _ in range(_K - 1):
        m = jnp.max(p, axis=1, keepdims=True)
        sel = p == m
        p = jnp.where(sel, pm, p)
        pm = jnp.where(sel, _NEG, pm)
        bv.append(m)
    cv = jnp.concatenate([runv_ref[...]] + bv, axis=1)
    nv = []
    for _ in range(_K - 1):
        m = jnp.max(cv, axis=1, keepdims=True)
        cv = jnp.where(cv == m, _NEG, cv)
        nv.append(m)
    runv_ref[...] = jnp.concatenate(nv, axis=1)
    tau_ref[...] = runv_ref[:, _K - 2:_K - 1]


def _stage2_agg_body(g_ref, db_ref, tau_ref, w_ref):
    b = pl.program_id(1)
    s = lax.dot_general(g_ref[...], db_ref[...], (((1,), (1,)), ((), ())),
                        precision=_PREC, preferred_element_type=jnp.float32)
    col = lax.broadcasted_iota(jnp.int32, (_RT, _BN), 1) + b * _BN
    sm = jnp.where((s >= tau_ref[...]) & (col < _N), s, 0.0)
    p = lax.dot_general(sm, db_ref[...], (((1,), (0,)), ((), ())),
                        precision=_PREC_ACC,
                        preferred_element_type=jnp.float32)

    @pl.when(b == 0)
    def _():
        w_ref[...] = jnp.zeros((_RT, _D), jnp.float32)

    w_ref[...] += p


_NW = 32
_RPW = _Q * _M // _NW


@functools.partial(
    pl.kernel,
    mesh=plsc.VectorSubcoreMesh(core_axis_name="c", subcore_axis_name="s"),
    out_type=jax.ShapeDtypeStruct((_Q * _M, _D), jnp.float32),
    scratch_types=[pltpu.VMEM((_RPW,), jnp.int32),
                   pltpu.VMEM((_RPW, _D), jnp.float32),
                   pltpu.SemaphoreType.DMA],
)
def _sc_gather(idx_hbm, db_hbm, out_hbm, idx_v, rows_v, sem):
    wid = lax.axis_index("s") * 2 + lax.axis_index("c")
    base = wid * _RPW
    pltpu.sync_copy(idx_hbm.at[pl.ds(base, _RPW)], idx_v)
    pltpu.async_copy(db_hbm.at[idx_v], rows_v, sem).wait()
    pltpu.sync_copy(rows_v, out_hbm.at[pl.ds(base, _RPW)])


def _final_body(q_ref, g_ref, w2_ref, ids_ref, selids_ref, scores_ref):
    q = q_ref[...]
    t = q
    for j in range(_K - 1):
        t = jnp.maximum(t, g_ref[:, j, :])
    tn = t * lax.rsqrt(jnp.maximum(jnp.sum(t * t, axis=1, keepdims=True),
                                   1e-24))
    def _b(x):
        return x.astype(jnp.bfloat16).astype(jnp.float32)
    qb = _b(q)
    tb = _b(tn)
    svals = []
    for mi in range(_M):
        w = q + w2_ref[:, mi, :]
        rn = w * lax.rsqrt(jnp.maximum(jnp.sum(w * w, axis=1, keepdims=True),
                                       1e-24))
        rb = _b(rn)
        s1 = jnp.sum(rb * qb, axis=1, keepdims=True)
        s2 = jnp.sum(rb * tb, axis=1, keepdims=True)
        svals.append(0.5 * (s1 + s2))
    sc = jnp.concatenate(svals, axis=1)
    colm = lax.broadcasted_iota(jnp.int32, (_Q, _M), 1)
    ids = ids_ref[...]
    outv, outi = [], []
    for _ in range(_TOP_X):
        m = jnp.max(sc, axis=1, keepdims=True)
        im = jnp.min(jnp.where(sc == m, colm, _IMAX), axis=1, keepdims=True)
        sel = jnp.sum(jnp.where(colm == im, ids, 0), axis=1, keepdims=True)
        sc = jnp.where(colm == im, _NEG, sc)
        outv.append(m)
        outi.append(sel)
    scores_ref[...] = jnp.concatenate(outv, axis=1)
    selids_ref[...] = jnp.concatenate(outi, axis=1)


def kernel(query_features, db_keys, db_ids):
    del db_ids
    q = query_features
    db = jnp.pad(db_keys, ((0, _NPAD - _N), (0, 0)))

    ids = pl.pallas_call(
        _stage1_body,
        grid=(_NB,),
        in_specs=[
            pl.BlockSpec((_Q, _D), lambda b: (0, 0)),
            pl.BlockSpec((_BN, _D), lambda b: (b, 0)),
        ],
        out_specs=pl.BlockSpec((_Q, _M), lambda b: (0, 0)),
        out_shape=jax.ShapeDtypeStruct((_Q, _M), jnp.int32),
        scratch_shapes=[pltpu.VMEM((_Q, _M), jnp.float32),
                        pltpu.VMEM((_Q, _M), jnp.int32)],
    )(q, db)

    flat_ids = ids.reshape(_Q * _M)
    g = _sc_gather(flat_ids, db_keys)

    nrt = _Q * _M // _RT
    tau = pl.pallas_call(
        _stage2_tau_body,
        grid=(nrt, _NB),
        in_specs=[
            pl.BlockSpec((_RT, _D), lambda r, b: (r, 0)),
            pl.BlockSpec((_BN, _D), lambda r, b: (b, 0)),
        ],
        out_specs=pl.BlockSpec((_RT, 1), lambda r, b: (r, 0)),
        out_shape=jax.ShapeDtypeStruct((_Q * _M, 1), jnp.float32),
        scratch_shapes=[pltpu.VMEM((_RT, _K - 1), jnp.float32)],
    )(g, db)

    w2 = pl.pallas_call(
        _stage2_agg_body,
        grid=(nrt, _NB),
        in_specs=[
            pl.BlockSpec((_RT, _D), lambda r, b: (r, 0)),
            pl.BlockSpec((_BN, _D), lambda r, b: (b, 0)),
            pl.BlockSpec((_RT, 1), lambda r, b: (r, 0)),
        ],
        out_specs=pl.BlockSpec((_RT, _D), lambda r, b: (r, 0)),
        out_shape=jax.ShapeDtypeStruct((_Q * _M, _D), jnp.float32),
    )(g, db, tau)

    selids, scores = pl.pallas_call(
        _final_body,
        in_specs=[
            pl.BlockSpec((_Q, _D), lambda: (0, 0)),
            pl.BlockSpec((_Q, _M, _D), lambda: (0, 0, 0)),
            pl.BlockSpec((_Q, _M, _D), lambda: (0, 0, 0)),
            pl.BlockSpec((_Q, _M), lambda: (0, 0)),
        ],
        out_specs=[pl.BlockSpec((_Q, _TOP_X), lambda: (0, 0)),
                   pl.BlockSpec((_Q, _TOP_X), lambda: (0, 0))],
        out_shape=[jax.ShapeDtypeStruct((_Q, _TOP_X), jnp.int32),
                   jax.ShapeDtypeStruct((_Q, _TOP_X), jnp.float32)],
    )(q, g.reshape(_Q, _M, _D), w2.reshape(_Q, _M, _D), ids)

    return selids, scores
